# Initial kernel scaffold; baseline (speedup 1.0000x reference)
import jax
import jax.numpy as jnp
from jax.experimental import pallas as pl


def kernel(token_id, s, tok_emb, Wq, col_id, Wk_in, Wv_in, w_decay, b_decay, input_E_bias, Wk_out, Wv_out, motor_query, input_positions):
    raise NotImplementedError("write your pallas kernel here")


# speedup vs baseline: 1.3390x; 1.3390x over previous
import functools
import jax
import jax.numpy as jnp
from jax.experimental import pallas as pl
from jax.experimental.pallas import tpu as pltpu


def _gather_rows_kernel(tok_ref, row_ref, out_ref):
    out_ref[...] = row_ref[...]


def _prologue_kernel(h_ref, Wq_ref, col_id_ref, Wk_in_ref, Wv_in_ref,
                     w_decay_ref, b_decay_ref, bias_ref, Wk_out_ref,
                     mq_ref, ip_ref,
                     v_ref, cntN_ref, g_ref, u_ref, a_ref,
                     *, B, N, D_s, H, Dq, N_in):
    h = h_ref[...]
    q = jnp.dot(h, Wq_ref[...], preferred_element_type=jnp.float32)
    iota_n = jax.lax.broadcasted_iota(jnp.int32, (N_in, N), 1)
    P = (iota_n == ip_ref[...]).astype(jnp.float32)
    in_ids = jnp.dot(P, col_id_ref[...], preferred_element_type=jnp.float32)
    keys = jnp.dot(in_ids, Wk_in_ref[...], preferred_element_type=jnp.float32)
    inv_sqrt_dq = 1.0 / (Dq ** 0.5)
    ji = jax.lax.broadcasted_iota(jnp.int32, (B, N_in), 1)
    cnt_in = jnp.zeros((B, N_in), dtype=jnp.float32)
    for hh in range(H):
        qh = q[:, hh * Dq:(hh + 1) * Dq]
        sc = jax.lax.dot_general(qh, keys, (((1,), (1,)), ((), ())),
                                 preferred_element_type=jnp.float32) * inv_sqrt_dq
        sc = sc + bias_ref[hh:hh + 1, :]
        mx = jnp.max(sc, axis=1, keepdims=True)
        sel = jnp.where(sc == mx, ji, N_in)
        jloc = jnp.min(sel, axis=1, keepdims=True)
        cnt_in = cnt_in + (ji == jloc).astype(jnp.float32)
    cntN_ref[...] = jnp.dot(cnt_in, P, preferred_element_type=jnp.float32)
    v = jnp.dot(h, Wv_in_ref[...], preferred_element_type=jnp.float32)
    v_ref[...] = v
    a_ref[...] = jax.nn.sigmoid(
        jnp.dot(col_id_ref[...], w_decay_ref[...],
                preferred_element_type=jnp.float32) + b_decay_ref[...])
    u = jax.lax.dot_general(mq_ref[...], Wk_out_ref[...], (((1,), (1,)), ((), ())),
                            preferred_element_type=jnp.float32)
    u_ref[...] = u
    g_ref[...] = jax.lax.dot_general(v, u, (((1,), (1,)), ((), ())),
                                     preferred_element_type=jnp.float32)


def _dot_u_kernel(s_ref, u_ref, t_ref):
    sb = s_ref[...]
    u = u_ref[...]
    u3 = jnp.reshape(u, (1, 1, u.shape[1]))
    t_ref[...] = jnp.sum(sb * u3, axis=2)


def _softmax_kernel(t_ref, a_ref, cntN_ref, g_ref, w_ref, cw_ref, *, D_s):
    t = t_ref[...]
    a = a_ref[...]
    cntN = cntN_ref[...]
    g = g_ref[...]
    logit = (t * a + cntN * g) * (1.0 / (D_s ** 0.5))
    mx = jnp.max(logit, axis=1, keepdims=True)
    e = jnp.exp(logit - mx)
    att = e / jnp.sum(e, axis=1, keepdims=True)
    w_ref[...] = att * a
    cw_ref[...] = jnp.sum(att * cntN, axis=1, keepdims=True)


def _weighted_sum_kernel(w_ref, s_ref, o_ref):
    j = pl.program_id(1)
    wb = w_ref[...]
    sb = s_ref[...]
    part = jnp.einsum('bn,bnd->bd', wb, sb,
                      preferred_element_type=jnp.float32)

    @pl.when(j == 0)
    def _init():
        o_ref[...] = part

    @pl.when(j > 0)
    def _acc():
        o_ref[...] += part


def _motor_kernel(m_ref, v_ref, cw_ref, Wv_out_ref, motor_ref):
    m_tot = m_ref[...] + cw_ref[...] * v_ref[...]
    motor0 = jnp.dot(m_tot, Wv_out_ref[...], preferred_element_type=jnp.float32)
    ms = jnp.mean(motor0 * motor0, axis=1, keepdims=True)
    motor_ref[...] = motor0 * jax.lax.rsqrt(ms + 1e-6)


def _logits_kernel(motor_ref, te_ref, out_ref):
    out_ref[...] = jax.lax.dot_general(
        motor_ref[...], te_ref[...], (((1,), (1,)), ((), ())),
        preferred_element_type=jnp.float32)


def kernel(token_id, s, tok_emb, Wq, col_id, Wk_in, Wv_in, w_decay, b_decay,
           input_E_bias, Wk_out, Wv_out, motor_query, input_positions):
    B, N, D_s = s.shape
    V = tok_emb.shape[0]
    D_id = col_id.shape[1]
    H, N_in = input_E_bias.shape
    Dq = Wk_in.shape[1]
    f32 = jnp.float32

    h = pl.pallas_call(
        _gather_rows_kernel,
        grid_spec=pltpu.PrefetchScalarGridSpec(
            num_scalar_prefetch=1,
            grid=(B,),
            in_specs=[pl.BlockSpec((1, 1, D_s), lambda i, tok: (tok[i], 0, 0))],
            out_specs=pl.BlockSpec((1, 1, D_s), lambda i, tok: (i, 0, 0)),
        ),
        out_shape=jax.ShapeDtypeStruct((B, 1, D_s), f32),
    )(token_id.astype(jnp.int32), tok_emb.reshape(V, 1, D_s))
    h = h.reshape(B, D_s)

    mq2 = motor_query.reshape(1, D_s)
    bd2 = b_decay.reshape(1, 1)
    ip2 = input_positions.astype(jnp.int32).reshape(N_in, 1)
    v, cntN, g, u_row, a_col = pl.pallas_call(
        functools.partial(_prologue_kernel, B=B, N=N, D_s=D_s, H=H, Dq=Dq,
                          N_in=N_in),
        out_shape=(
            jax.ShapeDtypeStruct((B, D_s), f32),
            jax.ShapeDtypeStruct((B, N), f32),
            jax.ShapeDtypeStruct((B, 1), f32),
            jax.ShapeDtypeStruct((1, D_s), f32),
            jax.ShapeDtypeStruct((N, 1), f32),
        ),
    )(h, Wq, col_id, Wk_in, Wv_in, w_decay, bd2, input_E_bias, Wk_out,
      mq2, ip2)
    a_row = a_col.reshape(1, N)

    bb, nb = 8, 512
    t = pl.pallas_call(
        _dot_u_kernel,
        grid=(B // bb, N // nb),
        in_specs=[
            pl.BlockSpec((bb, nb, D_s), lambda i, j: (i, j, 0)),
            pl.BlockSpec((1, D_s), lambda i, j: (0, 0)),
        ],
        out_specs=pl.BlockSpec((bb, nb), lambda i, j: (i, j)),
        out_shape=jax.ShapeDtypeStruct((B, N), f32),
    )(s, u_row)

    w, cw = pl.pallas_call(
        functools.partial(_softmax_kernel, D_s=D_s),
        out_shape=(
            jax.ShapeDtypeStruct((B, N), f32),
            jax.ShapeDtypeStruct((B, 1), f32),
        ),
    )(t, a_row, cntN, g)

    m = pl.pallas_call(
        _weighted_sum_kernel,
        grid=(B // bb, N // nb),
        in_specs=[
            pl.BlockSpec((bb, nb), lambda i, j: (i, j)),
            pl.BlockSpec((bb, nb, D_s), lambda i, j: (i, j, 0)),
        ],
        out_specs=pl.BlockSpec((bb, D_s), lambda i, j: (i, 0)),
        out_shape=jax.ShapeDtypeStruct((B, D_s), f32),
    )(w, s)

    motor = pl.pallas_call(
        _motor_kernel,
        out_shape=jax.ShapeDtypeStruct((B, D_s), f32),
    )(m, v, cw, Wv_out)

    vb = 2048
    logits = pl.pallas_call(
        _logits_kernel,
        grid=(V // vb,),
        in_specs=[
            pl.BlockSpec((B, D_s), lambda j: (0, 0)),
            pl.BlockSpec((vb, D_s), lambda j: (j, 0)),
        ],
        out_specs=pl.BlockSpec((B, vb), lambda j: (0, j)),
        out_shape=jax.ShapeDtypeStruct((B, V), f32),
    )(motor, tok_emb)
    return logits
